# Initial kernel scaffold; baseline (speedup 1.0000x reference)
import jax
import jax.numpy as jnp
from jax.experimental import pallas as pl


def kernel(inputs, edge_index, W0, b0, W1, b1):
    raise NotImplementedError("write your pallas kernel here")


# speedup vs baseline: 10.2468x; 10.2468x over previous
import functools

import jax
import jax.numpy as jnp
from jax import lax
from jax.experimental import pallas as pl
from jax.experimental.pallas import tpu as pltpu
from jax.experimental.pallas import tpu_sc as plsc

N = 10000
E = 320000
D = 128
H = 64
C = 40
K = 10
ALPHA = 0.1

W = 48
NPAD = 10112
DUMMY = N
NC, NS = 2, 16
NWORK = NC * NS
CH = 128
NCHUNK = 80
EPT = CH * NCHUNK
EPAD = EPT * NWORK
RPT = NPAD // NS

_sc_mesh = plsc.VectorSubcoreMesh(
    core_axis_name="c", subcore_axis_name="s", num_cores=NC, num_subcores=NS
)


def _deg_body(cidx_hbm, out_hbm, ones_v, zeros_v, cslab_v, acc_sh):
    cid = lax.axis_index("c")
    sid = lax.axis_index("s")
    wid = cid * NS + sid

    @pl.loop(0, CH)
    def _(i):
        ones_v[i, :] = jnp.full((16,), 1.0, jnp.float32)

    @pl.loop(0, RPT)
    def _(i):
        zeros_v[i, :] = jnp.zeros((16,), jnp.float32)

    pltpu.sync_copy(zeros_v, acc_sh.at[pl.ds(sid * RPT, RPT)])
    pltpu.sync_copy(cidx_hbm.at[wid], cslab_v)
    plsc.subcore_barrier()

    @pl.loop(0, NCHUNK)
    def _(j):
        pltpu.sync_copy(ones_v, acc_sh.at[cslab_v.at[j]], add=True)

    plsc.subcore_barrier()
    pltpu.sync_copy(
        acc_sh.at[pl.ds(sid * RPT, RPT)], out_hbm.at[cid, pl.ds(sid * RPT, RPT)]
    )


@functools.partial(
    pl.kernel,
    out_type=jax.ShapeDtypeStruct((NC, NPAD, 16), jnp.float32),
    mesh=_sc_mesh,
    compiler_params=pltpu.CompilerParams(use_tc_tiling_on_sc=False),
    scratch_types=[
        pltpu.VMEM((CH, 16), jnp.float32),
        pltpu.VMEM((RPT, 16), jnp.float32),
        pltpu.VMEM((NCHUNK, CH), jnp.int32),
        pltpu.VMEM_SHARED((NPAD, 16), jnp.float32),
    ],
)
def _deg_sc(cidx_hbm, out_hbm, ones_v, zeros_v, cslab_v, acc_sh):
    _deg_body(cidx_hbm, out_hbm, ones_v, zeros_v, cslab_v, acc_sh)


def _scat_body(y_hbm, ridx_hbm, cidx_hbm, out_hbm,
               rslab_v, cslab_v, buf_v, zeros_v, acc_sh, sem):
    cid = lax.axis_index("c")
    sid = lax.axis_index("s")
    wid = cid * NS + sid

    @pl.loop(0, RPT)
    def _(i):
        for c in range(W // 16):
            zeros_v[i, pl.ds(c * 16, 16)] = jnp.zeros((16,), jnp.float32)

    pltpu.sync_copy(zeros_v, acc_sh.at[pl.ds(sid * RPT, RPT)])
    pltpu.sync_copy(ridx_hbm.at[wid], rslab_v)
    pltpu.sync_copy(cidx_hbm.at[wid], cslab_v)
    plsc.subcore_barrier()

    @pl.loop(0, NCHUNK)
    def _(j):
        pltpu.async_copy(y_hbm.at[rslab_v.at[j]], buf_v, sem).wait()
        pltpu.sync_copy(buf_v, acc_sh.at[cslab_v.at[j]], add=True)

    plsc.subcore_barrier()
    pltpu.sync_copy(
        acc_sh.at[pl.ds(sid * RPT, RPT)], out_hbm.at[cid, pl.ds(sid * RPT, RPT)]
    )


@functools.partial(
    pl.kernel,
    out_type=jax.ShapeDtypeStruct((NC, NPAD, W), jnp.float32),
    mesh=_sc_mesh,
    compiler_params=pltpu.CompilerParams(use_tc_tiling_on_sc=False),
    scratch_types=[
        pltpu.VMEM((NCHUNK, CH), jnp.int32),
        pltpu.VMEM((NCHUNK, CH), jnp.int32),
        pltpu.VMEM((CH, W), jnp.float32),
        pltpu.VMEM((RPT, W), jnp.float32),
        pltpu.VMEM_SHARED((NPAD, W), jnp.float32),
        pltpu.SemaphoreType.DMA,
    ],
)
def _scat_sc(y_hbm, ridx_hbm, cidx_hbm, out_hbm,
             rslab_v, cslab_v, buf_v, zeros_v, acc_sh, sem):
    _scat_body(y_hbm, ridx_hbm, cidx_hbm, out_hbm,
               rslab_v, cslab_v, buf_v, zeros_v, acc_sh, sem)


def _mlp_body(xp_ref, w0_ref, b0_ref, w1_ref, b1_ref, h_ref):
    g = jnp.dot(xp_ref[...], w0_ref[...], preferred_element_type=jnp.float32)
    g = jnp.maximum(g + b0_ref[...], 0.0)
    h_ref[...] = (
        jnp.dot(g, w1_ref[...], preferred_element_type=jnp.float32) + b1_ref[...]
    )


def _mlp_tc(xp, w0, b0r, w1p, b1p):
    return pl.pallas_call(
        _mlp_body,
        out_shape=jax.ShapeDtypeStruct((NPAD, W), jnp.float32),
    )(xp, w0, b0r, w1p, b1p)


def _prep_body(h_ref, degp_ref, dinv_ref, y0_ref):
    deg = degp_ref[0, :, 0:1] + degp_ref[1, :, 0:1] + 1.0
    dinv = jnp.broadcast_to(lax.rsqrt(deg), (NPAD, W))
    dinv_ref[...] = dinv
    y0_ref[...] = dinv * h_ref[...]


def _prep_tc(h, degp):
    return pl.pallas_call(
        _prep_body,
        out_shape=(
            jax.ShapeDtypeStruct((NPAD, W), jnp.float32),
            jax.ShapeDtypeStruct((NPAD, W), jnp.float32),
        ),
    )(h, degp)


def _combine_body(sp_ref, x_ref, h_ref, dinv_ref, xn_ref, yn_ref):
    dinv = dinv_ref[...]
    s = sp_ref[0] + sp_ref[1]
    xn = (1.0 - ALPHA) * (dinv * s + dinv * dinv * x_ref[...]) + ALPHA * h_ref[...]
    xn_ref[...] = xn
    yn_ref[...] = dinv * xn


def _combine_tc(sp, x, h, dinv):
    return pl.pallas_call(
        _combine_body,
        out_shape=(
            jax.ShapeDtypeStruct((NPAD, W), jnp.float32),
            jax.ShapeDtypeStruct((NPAD, W), jnp.float32),
        ),
    )(sp, x, h, dinv)


def _last_body(sp_ref, x_ref, h_ref, dinv_ref, out_ref):
    dinv = dinv_ref[...]
    s = sp_ref[0] + sp_ref[1]
    xn = (1.0 - ALPHA) * (dinv * s + dinv * dinv * x_ref[...]) + ALPHA * h_ref[...]
    mask = lax.broadcasted_iota(jnp.int32, (NPAD, W), 1) < C
    xm = jnp.where(mask, xn, -jnp.inf)
    m = jnp.max(xm, axis=1, keepdims=True)
    e = jnp.where(mask, jnp.exp(xm - m), 0.0)
    lse = jnp.log(jnp.sum(e, axis=1, keepdims=True)) + m
    out_ref[...] = xn - lse


def _last_tc(sp, x, h, dinv):
    return pl.pallas_call(
        _last_body,
        out_shape=jax.ShapeDtypeStruct((NPAD, W), jnp.float32),
    )(sp, x, h, dinv)


def kernel(inputs, edge_index, W0, b0, W1, b1):
    row = edge_index[0].astype(jnp.int32)
    col = edge_index[1].astype(jnp.int32)
    npad_e = EPAD - E
    ridx = jnp.concatenate([row, jnp.zeros((npad_e,), jnp.int32)])
    cidx = jnp.concatenate([col, jnp.full((npad_e,), DUMMY, jnp.int32)])
    ridx = ridx.reshape(NWORK, NCHUNK, CH)
    cidx = cidx.reshape(NWORK, NCHUNK, CH)

    xp = jnp.zeros((NPAD, D), jnp.float32).at[:N].set(inputs)
    w1p = jnp.zeros((H, W), jnp.float32).at[:, :C].set(W1)
    b1p = jnp.zeros((1, W), jnp.float32).at[0, :C].set(b1)
    b0r = b0.reshape(1, H)

    degp = _deg_sc(cidx)
    h = _mlp_tc(xp, W0, b0r, w1p, b1p)
    dinv, y = _prep_tc(h, degp)
    x = h
    for _ in range(K - 1):
        sp = _scat_sc(y, ridx, cidx)
        x, y = _combine_tc(sp, x, h, dinv)
    sp = _scat_sc(y, ridx, cidx)
    out = _last_tc(sp, x, h, dinv)
    return out[:N, :C]


# speedup vs baseline: 12.4108x; 1.2112x over previous
import functools

import jax
import jax.numpy as jnp
from jax import lax
from jax.experimental import pallas as pl
from jax.experimental.pallas import tpu as pltpu
from jax.experimental.pallas import tpu_sc as plsc

N = 10000
E = 320000
D = 128
H = 64
C = 40
K = 10
ALPHA = 0.1

W = 48
NPAD = 10112
DUMMY = N
NC, NS = 2, 16
NWORK = NC * NS
CH = 128
NCHUNK = 80
EPT = CH * NCHUNK
EPAD = EPT * NWORK
RPT = NPAD // NS

KB = 5
NBATCH = NCHUNK // KB

_sc_mesh = plsc.VectorSubcoreMesh(
    core_axis_name="c", subcore_axis_name="s", num_cores=NC, num_subcores=NS
)


def _deg_body(cidx_hbm, ones_hbm, z16_hbm, out_hbm, ones_v, cslab_v, acc_sh, sem):
    cid = lax.axis_index("c")
    sid = lax.axis_index("s")
    wid = cid * NS + sid

    pltpu.sync_copy(ones_hbm, ones_v)
    pltpu.sync_copy(
        z16_hbm.at[pl.ds(sid * RPT, RPT)], acc_sh.at[pl.ds(sid * RPT, RPT)]
    )
    pltpu.sync_copy(cidx_hbm.at[wid], cslab_v)
    plsc.subcore_barrier()

    @pl.loop(0, NCHUNK, step=16)
    def _(j):
        for b in range(16):
            pltpu.async_copy(ones_v, acc_sh.at[cslab_v.at[j + b]], sem, add=True)
        for b in range(16):
            pltpu.make_async_copy(ones_v, acc_sh.at[cslab_v.at[0]], sem).wait()

    plsc.subcore_barrier()
    pltpu.sync_copy(
        acc_sh.at[pl.ds(sid * RPT, RPT)], out_hbm.at[cid, pl.ds(sid * RPT, RPT)]
    )


@functools.partial(
    pl.kernel,
    out_type=jax.ShapeDtypeStruct((NC, NPAD, 16), jnp.float32),
    mesh=_sc_mesh,
    compiler_params=pltpu.CompilerParams(use_tc_tiling_on_sc=False),
    scratch_types=[
        pltpu.VMEM((CH, 16), jnp.float32),
        pltpu.VMEM((NCHUNK, CH), jnp.int32),
        pltpu.VMEM_SHARED((NPAD, 16), jnp.float32),
        pltpu.SemaphoreType.DMA,
    ],
)
def _deg_sc(cidx_hbm, ones_hbm, z16_hbm, out_hbm, ones_v, cslab_v, acc_sh, sem):
    _deg_body(cidx_hbm, ones_hbm, z16_hbm, out_hbm, ones_v, cslab_v, acc_sh, sem)


def _scat_body(y_hbm, ridx_hbm, cidx_hbm, z_hbm, out_hbm,
               rslab_v, cslab_v, bufs, acc_sh, gsems, ssems):
    cid = lax.axis_index("c")
    sid = lax.axis_index("s")
    wid = cid * NS + sid
    banks = (bufs[:KB], bufs[KB:])

    def fire_g(c0, k, sem):
        for b in range(KB):
            pltpu.async_copy(y_hbm.at[rslab_v.at[c0 + b]], banks[k][b], sem)

    def drain_g(k, sem):
        for b in range(KB):
            pltpu.make_async_copy(y_hbm.at[rslab_v.at[0]], banks[k][b], sem).wait()

    def fire_s(c0, k, sem):
        for b in range(KB):
            pltpu.async_copy(
                banks[k][b], acc_sh.at[cslab_v.at[c0 + b]], sem, add=True
            )

    def drain_s(k, sem):
        for b in range(KB):
            pltpu.make_async_copy(banks[k][b], acc_sh.at[cslab_v.at[0]], sem).wait()

    pltpu.sync_copy(
        z_hbm.at[pl.ds(sid * RPT, RPT)], acc_sh.at[pl.ds(sid * RPT, RPT)]
    )
    pltpu.sync_copy(ridx_hbm.at[wid], rslab_v)
    pltpu.sync_copy(cidx_hbm.at[wid], cslab_v)
    plsc.subcore_barrier()

    fire_g(0, 0, gsems[0])

    @pl.loop(0, NBATCH, step=2)
    def _(i):
        c0 = i * KB
        drain_g(0, gsems[0])
        fire_s(c0, 0, ssems[0])
        fire_g(c0 + KB, 1, gsems[1])
        drain_s(0, ssems[0])
        drain_g(1, gsems[1])
        fire_s(c0 + KB, 1, ssems[1])

        @pl.when(i + 2 < NBATCH)
        def _():
            fire_g(c0 + 2 * KB, 0, gsems[0])

        drain_s(1, ssems[1])

    plsc.subcore_barrier()
    pltpu.sync_copy(
        acc_sh.at[pl.ds(sid * RPT, RPT)], out_hbm.at[cid, pl.ds(sid * RPT, RPT)]
    )


@functools.partial(
    pl.kernel,
    out_type=jax.ShapeDtypeStruct((NC, NPAD, W), jnp.float32),
    mesh=_sc_mesh,
    compiler_params=pltpu.CompilerParams(use_tc_tiling_on_sc=False),
    scratch_types=(
        [pltpu.VMEM((NCHUNK, CH), jnp.int32)] * 2
        + [pltpu.VMEM((CH, W), jnp.float32)] * (2 * KB)
        + [pltpu.VMEM_SHARED((NPAD, W), jnp.float32)]
        + [pltpu.SemaphoreType.DMA] * 4
    ),
)
def _scat_sc(y_hbm, ridx_hbm, cidx_hbm, z_hbm, out_hbm, *rest):
    rslab_v, cslab_v = rest[0], rest[1]
    bufs = list(rest[2:2 + 2 * KB])
    acc_sh = rest[2 + 2 * KB]
    sems = rest[3 + 2 * KB:]
    _scat_body(y_hbm, ridx_hbm, cidx_hbm, z_hbm, out_hbm,
               rslab_v, cslab_v, bufs, acc_sh,
               (sems[0], sems[1]), (sems[2], sems[3]))


def _mlp_body(xp_ref, w0_ref, b0_ref, w1_ref, b1_ref, h_ref):
    g = jnp.dot(xp_ref[...], w0_ref[...], preferred_element_type=jnp.float32)
    g = jnp.maximum(g + b0_ref[...], 0.0)
    h_ref[...] = (
        jnp.dot(g, w1_ref[...], preferred_element_type=jnp.float32) + b1_ref[...]
    )


def _mlp_tc(xp, w0, b0r, w1p, b1p):
    return pl.pallas_call(
        _mlp_body,
        out_shape=jax.ShapeDtypeStruct((NPAD, W), jnp.float32),
    )(xp, w0, b0r, w1p, b1p)


def _prep_body(h_ref, degp_ref, dinv_ref, y0_ref):
    deg = degp_ref[0, :, 0:1] + degp_ref[1, :, 0:1] + 1.0
    dinv = jnp.broadcast_to(lax.rsqrt(deg), (NPAD, W))
    dinv_ref[...] = dinv
    y0_ref[...] = dinv * h_ref[...]


def _prep_tc(h, degp):
    return pl.pallas_call(
        _prep_body,
        out_shape=(
            jax.ShapeDtypeStruct((NPAD, W), jnp.float32),
            jax.ShapeDtypeStruct((NPAD, W), jnp.float32),
        ),
    )(h, degp)


def _combine_body(sp_ref, x_ref, h_ref, dinv_ref, xn_ref, yn_ref):
    dinv = dinv_ref[...]
    s = sp_ref[0] + sp_ref[1]
    xn = (1.0 - ALPHA) * (dinv * s + dinv * dinv * x_ref[...]) + ALPHA * h_ref[...]
    xn_ref[...] = xn
    yn_ref[...] = dinv * xn


def _combine_tc(sp, x, h, dinv):
    return pl.pallas_call(
        _combine_body,
        out_shape=(
            jax.ShapeDtypeStruct((NPAD, W), jnp.float32),
            jax.ShapeDtypeStruct((NPAD, W), jnp.float32),
        ),
    )(sp, x, h, dinv)


def _last_body(sp_ref, x_ref, h_ref, dinv_ref, out_ref):
    dinv = dinv_ref[...]
    s = sp_ref[0] + sp_ref[1]
    xn = (1.0 - ALPHA) * (dinv * s + dinv * dinv * x_ref[...]) + ALPHA * h_ref[...]
    mask = lax.broadcasted_iota(jnp.int32, (NPAD, W), 1) < C
    xm = jnp.where(mask, xn, -jnp.inf)
    m = jnp.max(xm, axis=1, keepdims=True)
    e = jnp.where(mask, jnp.exp(xm - m), 0.0)
    lse = jnp.log(jnp.sum(e, axis=1, keepdims=True)) + m
    out_ref[...] = xn - lse


def _last_tc(sp, x, h, dinv):
    return pl.pallas_call(
        _last_body,
        out_shape=jax.ShapeDtypeStruct((NPAD, W), jnp.float32),
    )(sp, x, h, dinv)


def kernel(inputs, edge_index, W0, b0, W1, b1):
    row = edge_index[0].astype(jnp.int32)
    col = edge_index[1].astype(jnp.int32)
    npad_e = EPAD - E
    ridx = jnp.concatenate([row, jnp.zeros((npad_e,), jnp.int32)])
    cidx = jnp.concatenate([col, jnp.full((npad_e,), DUMMY, jnp.int32)])
    ridx = ridx.reshape(NWORK, NCHUNK, CH)
    cidx = cidx.reshape(NWORK, NCHUNK, CH)

    xp = jnp.zeros((NPAD, D), jnp.float32).at[:N].set(inputs)
    w1p = jnp.zeros((H, W), jnp.float32).at[:, :C].set(W1)
    b1p = jnp.zeros((1, W), jnp.float32).at[0, :C].set(b1)
    b0r = b0.reshape(1, H)
    ones16 = jnp.ones((CH, 16), jnp.float32)
    z16 = jnp.zeros((NPAD, 16), jnp.float32)
    z48 = jnp.zeros((NPAD, W), jnp.float32)

    degp = _deg_sc(cidx, ones16, z16)
    h = _mlp_tc(xp, W0, b0r, w1p, b1p)
    dinv, y = _prep_tc(h, degp)
    x = h
    for _ in range(K - 1):
        sp = _scat_sc(y, ridx, cidx, z48)
        x, y = _combine_tc(sp, x, h, dinv)
    sp = _scat_sc(y, ridx, cidx, z48)
    out = _last_tc(sp, x, h, dinv)
    return out[:N, :C]
